# Initial kernel scaffold; baseline (speedup 1.0000x reference)
import jax
import jax.numpy as jnp
from jax.experimental import pallas as pl


def kernel(s, v, pos, edge_index, rbf_centers, rbf_widths, ps_w1, ps_b1, ps_w2, ps_b2, w_w1, w_b1, w_w2, w_b2):
    raise NotImplementedError("write your pallas kernel here")


# speedup vs baseline: 4.8726x; 4.8726x over previous
import functools
import math

import jax
import jax.numpy as jnp
from jax import lax
from jax.experimental import pallas as pl
from jax.experimental.pallas import tpu as pltpu

N = 10000
E = 320000
H = 128
R = 20
CUTOFF = 8.0

EDGE_BLK = 1280


def _edge_math_kernel(psrc_ref, pdst_ref, ssrc_ref,
                      cpad_ref, wpad_ref,
                      w_w1_ref, w_b1_ref, w_w2_ref, w_b2_ref,
                      ps_w1_ref, ps_b1_ref, ps_w2_ref, ps_b2_ref,
                      u_ref, ds_ref, dv1_ref, dv2_ref):
    r = pdst_ref[...] - psrc_ref[...]
    d2 = jnp.sum(r * r, axis=1, keepdims=True)
    d = jnp.maximum(jnp.sqrt(d2), 1e-6)
    u = r / d
    u_ref[...] = u
    x = d / CUTOFF
    env = jnp.where(x < 1.0, 0.5 * (jnp.cos(math.pi * x) + 1.0), 0.0)
    feat = jnp.exp(-wpad_ref[...] * (d - cpad_ref[...]) ** 2) * env
    h1 = jnp.dot(feat, w_w1_ref[...], preferred_element_type=jnp.float32)
    h1 = h1 + w_b1_ref[...]
    h1 = h1 * jax.nn.sigmoid(h1)
    W = jnp.dot(h1, w_w2_ref[...], preferred_element_type=jnp.float32)
    W = W + w_b2_ref[...]
    g1 = jnp.dot(ssrc_ref[...], ps_w1_ref[...], preferred_element_type=jnp.float32)
    g1 = g1 + ps_b1_ref[...]
    g1 = g1 * jax.nn.sigmoid(g1)
    phi = jnp.dot(g1, ps_w2_ref[...], preferred_element_type=jnp.float32)
    phi = phi + ps_b2_ref[...]
    sp = phi * W
    ds_ref[...] = sp[:, 0:H]
    dv1_ref[...] = sp[:, H:2 * H]
    dv2_ref[...] = sp[:, 2 * H:3 * H]


def _edge_math(psrc, pdst, ssrc, cpad, wpad, w_w1p, w_b1, w_w2, w_b2,
               ps_w1, ps_b1, ps_w2, ps_b2):
    grid = (E // EDGE_BLK,)
    eblk = lambda w: pl.BlockSpec((EDGE_BLK, w), lambda i: (i, 0))
    full = lambda a: pl.BlockSpec(a.shape, lambda i: (0,) * a.ndim)
    return pl.pallas_call(
        _edge_math_kernel,
        grid=grid,
        in_specs=[eblk(8), eblk(8), eblk(H),
                  full(cpad), full(wpad),
                  full(w_w1p), full(w_b1), full(w_w2), full(w_b2),
                  full(ps_w1), full(ps_b1), full(ps_w2), full(ps_b2)],
        out_specs=[eblk(8), eblk(H), eblk(H), eblk(H)],
        out_shape=[
            jax.ShapeDtypeStruct((E, 8), jnp.float32),
            jax.ShapeDtypeStruct((E, H), jnp.float32),
            jax.ShapeDtypeStruct((E, H), jnp.float32),
            jax.ShapeDtypeStruct((E, H), jnp.float32),
        ],
    )(psrc, pdst, ssrc, cpad, wpad, w_w1p, w_b1, w_w2, w_b2,
      ps_w1, ps_b1, ps_w2, ps_b2)


def kernel(s, v, pos, edge_index, rbf_centers, rbf_widths,
           ps_w1, ps_b1, ps_w2, ps_b2, w_w1, w_b1, w_w2, w_b2):
    src = edge_index[0]
    dst = edge_index[1]

    pos8 = jnp.pad(pos, ((0, 0), (0, 5)))
    cpad = jnp.full((1, H), 1e9, jnp.float32).at[0, :R].set(rbf_centers)
    wpad = jnp.ones((1, H), jnp.float32).at[0, :R].set(jnp.abs(rbf_widths))
    w_w1p = jnp.zeros((H, H), jnp.float32).at[:R, :].set(w_w1)

    psrc = pos8[src]
    pdst = pos8[dst]
    ssrc = s[src]

    u, dS, dV1, dV2 = _edge_math(
        psrc, pdst, ssrc, cpad, wpad, w_w1p,
        w_b1.reshape(1, H), w_w2, w_b2.reshape(1, 3 * H),
        ps_w1, ps_b1.reshape(1, H), ps_w2, ps_b2.reshape(1, 3 * H))

    vt = jnp.transpose(v, (2, 0, 1))
    s_new = s + jax.ops.segment_sum(dS, dst, num_segments=N)
    outs = []
    for c in range(3):
        mv_c = dV1 * vt[c][src] + u[:, c:c + 1] * dV2
        outs.append(vt[c] + jax.ops.segment_sum(mv_c, dst, num_segments=N))
    v_new = jnp.transpose(jnp.stack(outs, axis=0), (1, 2, 0))
    return (s_new, v_new)


# speedup vs baseline: 7.5816x; 1.5560x over previous
import functools
import math

import jax
import jax.numpy as jnp
from jax import lax
from jax.experimental import pallas as pl
from jax.experimental.pallas import tpu as pltpu
from jax.experimental.pallas import tpu_sc as plsc

N = 10000
E = 320000
H = 128
R = 20
CUTOFF = 8.0

EDGE_BLK = 1280

NW = 32
EPW = E // NW
GK = 400
GCHUNKS = EPW // GK

_sc_mesh = plsc.VectorSubcoreMesh(core_axis_name="c", subcore_axis_name="s")


@functools.partial(
    pl.kernel,
    out_type=[jax.ShapeDtypeStruct((E, H), jnp.float32),
              jax.ShapeDtypeStruct((E * 8,), jnp.float32)],
    mesh=_sc_mesh,
    scratch_types=[pltpu.VMEM((GK,), jnp.int32),
                   pltpu.VMEM((GK,), jnp.int32),
                   pltpu.VMEM((GK, H), jnp.float32),
                   pltpu.VMEM((GK * 8,), jnp.float32),
                   pltpu.VMEM((N,), jnp.float32),
                   pltpu.VMEM((N,), jnp.float32),
                   pltpu.VMEM((N,), jnp.float32),
                   pltpu.SemaphoreType.DMA],
    compiler_params=pltpu.CompilerParams(needs_layout_passes=False, use_tc_tiling_on_sc=False),
)
def _sc_gather(s_hbm, px_hbm, py_hbm, pz_hbm, src_hbm, dst_hbm, ssrc_out, r_out,
               idx_s, idx_d, srows, rbuf, pxv, pyv, pzv, sem1):
    cid = lax.axis_index("c")
    sid = lax.axis_index("s")
    wid = sid * 2 + cid
    base = wid * EPW
    pltpu.sync_copy(px_hbm, pxv)
    pltpu.sync_copy(py_hbm, pyv)
    pltpu.sync_copy(pz_hbm, pzv)
    ptabs = (pxv, pyv, pzv)
    iota = lax.iota(jnp.int32, 16)

    def body(j, carry):
        e0 = base + j * GK
        pltpu.sync_copy(src_hbm.at[pl.ds(e0, GK)], idx_s)
        pltpu.sync_copy(dst_hbm.at[pl.ds(e0, GK)], idx_d)
        cp = pltpu.async_copy(s_hbm.at[idx_s], srows, sem1)

        def grp(g, carry2):
            iv_s = idx_s[pl.ds(g * 16, 16)]
            iv_d = idx_d[pl.ds(g * 16, 16)]
            for c in range(3):
                pc = (plsc.load_gather(ptabs[c], [iv_d])
                      - plsc.load_gather(ptabs[c], [iv_s]))
                plsc.store_scatter(rbuf, [iota * 8 + (g * 128 + c)], pc)
            return carry2

        lax.fori_loop(0, GK // 16, grp, 0)
        cp.wait()
        pltpu.sync_copy(srows, ssrc_out.at[pl.ds(e0, GK)])
        pltpu.sync_copy(rbuf, r_out.at[pl.ds(e0 * 8, GK * 8)])
        return carry

    lax.fori_loop(0, GCHUNKS, body, 0)


SK = 80
EPT = E // 16
SCHUNKS = EPT // SK
NPT = N // 16


def _make_scatter(kind0, kind1):
    @functools.partial(
        pl.kernel,
        out_type=[jax.ShapeDtypeStruct((N, H), jnp.float32),
                  jax.ShapeDtypeStruct((N, H), jnp.float32)],
        mesh=_sc_mesh,
        scratch_types=[pltpu.VMEM((SK,), jnp.int32),
                       pltpu.VMEM((SK,), jnp.int32),
                       pltpu.VMEM((SK, H), jnp.float32),
                       pltpu.VMEM((SK, H), jnp.float32),
                       pltpu.VMEM((SK, H), jnp.float32),
                       pltpu.VMEM((SK * 8,), jnp.float32),
                       pltpu.VMEM_SHARED((N, H), jnp.float32),
                       pltpu.SemaphoreType.DMA],
        compiler_params=pltpu.CompilerParams(needs_layout_passes=False,
                                             use_tc_tiling_on_sc=False),
    )
    def _scatter(tab0, tab1, m1a_hbm, m1b_hbm, m2_hbm, r_hbm, src_hbm, dst_hbm,
                 out0, out1,
                 idx_s, idx_d, m1, m2, trows, rbuf, accum, sem1):
        cid = lax.axis_index("c")
        sid = lax.axis_index("s")

        def job(kind, tab, m1src, out):
            @pl.when(sid == 0)
            def _():
                pltpu.sync_copy(tab, accum)
            plsc.subcore_barrier()

            def body(j, carry):
                e0 = sid * EPT + j * SK
                pltpu.sync_copy(dst_hbm.at[pl.ds(e0, SK)], idx_d)
                pltpu.sync_copy(m1src.at[pl.ds(e0, SK)], m1)
                if kind is not None:
                    pltpu.sync_copy(src_hbm.at[pl.ds(e0, SK)], idx_s)
                    pltpu.sync_copy(m2_hbm.at[pl.ds(e0, SK)], m2)
                    pltpu.sync_copy(r_hbm.at[pl.ds(e0 * 8, SK * 8)], rbuf)
                    pltpu.async_copy(tab.at[idx_s], trows, sem1).wait()

                    def edge(i, carry2):
                        us = plsc.load_gather(
                            rbuf, [jnp.full((16,), i * 8 + kind, jnp.int32)])
                        for g in range(8):
                            sl = pl.ds(g * 16, 16)
                            m1[i, sl] = (m1[i, sl] * trows[i, sl]
                                         + us * m2[i, sl])
                        return carry2

                    lax.fori_loop(0, SK, edge, 0)
                pltpu.sync_copy(m1, accum.at[idx_d], add=True)
                return carry

            lax.fori_loop(0, SCHUNKS, body, 0)
            plsc.subcore_barrier()
            pltpu.sync_copy(accum.at[pl.ds(sid * NPT, NPT)],
                            out.at[pl.ds(sid * NPT, NPT)])

        @pl.when(cid == 0)
        def _():
            job(kind0, tab0, m1a_hbm, out0)

        @pl.when(cid == 1)
        def _():
            job(kind1, tab1, m1b_hbm, out1)

    return _scatter


_scatter_call_1 = _make_scatter(None, 0)
_scatter_call_2 = _make_scatter(1, 2)


def _edge_math_kernel(r_ref, ssrc_ref,
                      cpad_ref, wpad_ref,
                      w_w1_ref, w_b1_ref, w_w2_ref, w_b2_ref,
                      ps_w1_ref, ps_b1_ref, ps_w2_ref, ps_b2_ref,
                      ds_ref, dv1_ref, dv2_ref):
    mask = (lax.broadcasted_iota(jnp.int32, (1, 8), 1) < 3).astype(jnp.float32)
    r = r_ref[...] * mask
    d2 = jnp.sum(r * r, axis=1, keepdims=True)
    d = jnp.maximum(jnp.sqrt(d2), 1e-6)
    x = d / CUTOFF
    env = jnp.where(x < 1.0, 0.5 * (jnp.cos(math.pi * x) + 1.0), 0.0)
    feat = jnp.exp(-wpad_ref[...] * (d - cpad_ref[...]) ** 2) * env
    h1 = jnp.dot(feat, w_w1_ref[...], preferred_element_type=jnp.float32)
    h1 = h1 + w_b1_ref[...]
    h1 = h1 * jax.nn.sigmoid(h1)
    W = jnp.dot(h1, w_w2_ref[...], preferred_element_type=jnp.float32)
    W = W + w_b2_ref[...]
    g1 = jnp.dot(ssrc_ref[...], ps_w1_ref[...], preferred_element_type=jnp.float32)
    g1 = g1 + ps_b1_ref[...]
    g1 = g1 * jax.nn.sigmoid(g1)
    phi = jnp.dot(g1, ps_w2_ref[...], preferred_element_type=jnp.float32)
    phi = phi + ps_b2_ref[...]
    sp = phi * W
    ds_ref[...] = sp[:, 0:H]
    dv1_ref[...] = sp[:, H:2 * H]
    dv2_ref[...] = sp[:, 2 * H:3 * H] / d


def _edge_math(r8, ssrc, cpad, wpad, w_w1p, w_b1, w_w2, w_b2,
               ps_w1, ps_b1, ps_w2, ps_b2):
    grid = (E // EDGE_BLK,)
    eblk = lambda w: pl.BlockSpec((EDGE_BLK, w), lambda i: (i, 0))
    full = lambda a: pl.BlockSpec(a.shape, lambda i: (0,) * a.ndim)
    return pl.pallas_call(
        _edge_math_kernel,
        grid=grid,
        in_specs=[eblk(8), eblk(H),
                  full(cpad), full(wpad),
                  full(w_w1p), full(w_b1), full(w_w2), full(w_b2),
                  full(ps_w1), full(ps_b1), full(ps_w2), full(ps_b2)],
        out_specs=[eblk(H), eblk(H), eblk(H)],
        out_shape=[
            jax.ShapeDtypeStruct((E, H), jnp.float32),
            jax.ShapeDtypeStruct((E, H), jnp.float32),
            jax.ShapeDtypeStruct((E, H), jnp.float32),
        ],
    )(r8, ssrc, cpad, wpad, w_w1p, w_b1, w_w2, w_b2,
      ps_w1, ps_b1, ps_w2, ps_b2)


def kernel(s, v, pos, edge_index, rbf_centers, rbf_widths,
           ps_w1, ps_b1, ps_w2, ps_b2, w_w1, w_b1, w_w2, w_b2):
    src = edge_index[0]
    dst = edge_index[1]

    px, py, pz = pos[:, 0], pos[:, 1], pos[:, 2]
    cpad = jnp.full((1, H), 1e9, jnp.float32).at[0, :R].set(rbf_centers)
    wpad = jnp.ones((1, H), jnp.float32).at[0, :R].set(jnp.abs(rbf_widths))
    w_w1p = jnp.zeros((H, H), jnp.float32).at[:R, :].set(w_w1)

    ssrc, r_flat = _sc_gather(s, px, py, pz, src, dst)
    r8 = r_flat.reshape(E, 8)

    dS, dV1, dV2s = _edge_math(
        r8, ssrc, cpad, wpad, w_w1p,
        w_b1.reshape(1, H), w_w2, w_b2.reshape(1, 3 * H),
        ps_w1, ps_b1.reshape(1, H), ps_w2, ps_b2.reshape(1, 3 * H))

    vt = jnp.transpose(v, (2, 0, 1))
    s_new, vo0 = _scatter_call_1(s, vt[0], dS, dV1, dV2s, r_flat, src, dst)
    vo1, vo2 = _scatter_call_2(vt[1], vt[2], dV1, dV1, dV2s, r_flat, src, dst)
    v_new = jnp.transpose(jnp.stack([vo0, vo1, vo2], axis=0), (1, 2, 0))
    return (s_new, v_new)


# speedup vs baseline: 10.4577x; 1.3794x over previous
import functools
import math

import jax
import jax.numpy as jnp
from jax import lax
from jax.experimental import pallas as pl
from jax.experimental.pallas import tpu as pltpu
from jax.experimental.pallas import tpu_sc as plsc

N = 10000
E = 320000
H = 128
R = 20
CUTOFF = 8.0

EDGE_BLK = 1280

NW = 32
EPW = E // NW
GK = 400
GCHUNKS = EPW // GK

_sc_mesh = plsc.VectorSubcoreMesh(core_axis_name="c", subcore_axis_name="s")


@functools.partial(
    pl.kernel,
    out_type=[jax.ShapeDtypeStruct((E, H), jnp.float32),
              jax.ShapeDtypeStruct((E * 8,), jnp.float32)],
    mesh=_sc_mesh,
    scratch_types=[pltpu.VMEM((GK,), jnp.int32),
                   pltpu.VMEM((GK,), jnp.int32),
                   pltpu.VMEM((GK, H), jnp.float32),
                   pltpu.VMEM((GK * 8,), jnp.float32),
                   pltpu.VMEM((N,), jnp.float32),
                   pltpu.VMEM((N,), jnp.float32),
                   pltpu.VMEM((N,), jnp.float32),
                   pltpu.SemaphoreType.DMA],
    compiler_params=pltpu.CompilerParams(needs_layout_passes=False, use_tc_tiling_on_sc=False),
)
def _sc_gather(s_hbm, px_hbm, py_hbm, pz_hbm, src_hbm, dst_hbm, ssrc_out, r_out,
               idx_s, idx_d, srows, rbuf, pxv, pyv, pzv, sem1):
    cid = lax.axis_index("c")
    sid = lax.axis_index("s")
    wid = sid * 2 + cid
    base = wid * EPW
    pltpu.sync_copy(px_hbm, pxv)
    pltpu.sync_copy(py_hbm, pyv)
    pltpu.sync_copy(pz_hbm, pzv)
    ptabs = (pxv, pyv, pzv)
    iota = lax.iota(jnp.int32, 16)

    def body(j, carry):
        e0 = base + j * GK
        pltpu.sync_copy(src_hbm.at[pl.ds(e0, GK)], idx_s)
        pltpu.sync_copy(dst_hbm.at[pl.ds(e0, GK)], idx_d)
        cp = pltpu.async_copy(s_hbm.at[idx_s], srows, sem1)

        def grp(g, carry2):
            iv_s = idx_s[pl.ds(g * 16, 16)]
            iv_d = idx_d[pl.ds(g * 16, 16)]
            for c in range(3):
                pc = (plsc.load_gather(ptabs[c], [iv_d])
                      - plsc.load_gather(ptabs[c], [iv_s]))
                plsc.store_scatter(rbuf, [iota * 8 + (g * 128 + c)], pc)
            return carry2

        lax.fori_loop(0, GK // 16, grp, 0)
        cp.wait()
        pltpu.sync_copy(srows, ssrc_out.at[pl.ds(e0, GK)])
        pltpu.sync_copy(rbuf, r_out.at[pl.ds(e0 * 8, GK * 8)])
        return carry

    lax.fori_loop(0, GCHUNKS, body, 0)


SK = 40
EPT = E // 16
SCHUNKS = EPT // SK
NPT = N // 16


def _make_scatter(kind0, kind1):
    @functools.partial(
        pl.kernel,
        out_type=[jax.ShapeDtypeStruct((N, H), jnp.float32),
                  jax.ShapeDtypeStruct((N, H), jnp.float32)],
        mesh=_sc_mesh,
        scratch_types=[pltpu.VMEM((SK,), jnp.int32),
                       pltpu.VMEM((SK,), jnp.int32),
                       pltpu.VMEM((SK,), jnp.int32),
                       pltpu.VMEM((SK,), jnp.int32),
                       pltpu.VMEM((SK, H), jnp.float32),
                       pltpu.VMEM((SK, H), jnp.float32),
                       pltpu.VMEM((SK, H), jnp.float32),
                       pltpu.VMEM((SK, H), jnp.float32),
                       pltpu.VMEM((SK, H), jnp.float32),
                       pltpu.VMEM((SK, H), jnp.float32),
                       pltpu.VMEM((SK * 8,), jnp.float32),
                       pltpu.VMEM((SK * 8,), jnp.float32),
                       pltpu.VMEM_SHARED((N, H), jnp.float32),
                       pltpu.SemaphoreType.DMA,
                       pltpu.SemaphoreType.DMA,
                       pltpu.SemaphoreType.DMA,
                       pltpu.SemaphoreType.DMA,
                       pltpu.SemaphoreType.DMA,
                       pltpu.SemaphoreType.DMA,
                       pltpu.SemaphoreType.DMA,
                       pltpu.SemaphoreType.DMA],
        compiler_params=pltpu.CompilerParams(needs_layout_passes=False,
                                             use_tc_tiling_on_sc=False),
    )
    def _scatter(tab0, tab1, m1a_hbm, m1b_hbm, m2_hbm, r_hbm, src_hbm, dst_hbm,
                 out0, out1,
                 ixs0, ixs1, ixd0, ixd1, m1_0, m1_1, m2_0, m2_1, tr0, tr1,
                 rb0, rb1, accum,
                 semi0, semi1, semf0, semf1, semg0, semg1, sema0, sema1):
        cid = lax.axis_index("c")
        sid = lax.axis_index("s")
        ixs = (ixs0, ixs1)
        ixd = (ixd0, ixd1)
        m1 = (m1_0, m1_1)
        m2 = (m2_0, m2_1)
        tr = (tr0, tr1)
        rb = (rb0, rb1)
        semi = (semi0, semi1)
        semf = (semf0, semf1)
        semg = (semg0, semg1)
        sema = (sema0, sema1)

        def job(kind, tab, m1src, out):
            @pl.when(sid == 0)
            def _():
                pltpu.sync_copy(tab, accum)
            plsc.subcore_barrier()

            def issue_fills(j, b):
                e0 = sid * EPT + j * SK
                pltpu.async_copy(dst_hbm.at[pl.ds(e0, SK)], ixd[b], semi[b])
                pltpu.async_copy(m1src.at[pl.ds(e0, SK)], m1[b], semf[b])
                if kind is not None:
                    pltpu.async_copy(src_hbm.at[pl.ds(e0, SK)], ixs[b], semi[b])
                    pltpu.async_copy(m2_hbm.at[pl.ds(e0, SK)], m2[b], semf[b])
                    pltpu.async_copy(r_hbm.at[pl.ds(e0 * 8, SK * 8)], rb[b],
                                     semf[b])

            def wait_idx(b):
                pltpu.make_async_copy(dst_hbm.at[pl.ds(0, SK)], ixd[b],
                                      semi[b]).wait()
                if kind is not None:
                    pltpu.make_async_copy(src_hbm.at[pl.ds(0, SK)], ixs[b],
                                          semi[b]).wait()

            def issue_gather(b):
                if kind is not None:
                    pltpu.async_copy(tab.at[ixs[b]], tr[b], semg[b])

            def wait_fills(b):
                pltpu.make_async_copy(m1src.at[pl.ds(0, SK)], m1[b],
                                      semf[b]).wait()
                if kind is not None:
                    pltpu.make_async_copy(m2_hbm.at[pl.ds(0, SK)], m2[b],
                                          semf[b]).wait()
                    pltpu.make_async_copy(r_hbm.at[pl.ds(0, SK * 8)], rb[b],
                                          semf[b]).wait()
                    pltpu.make_async_copy(tab.at[ixs[b]], tr[b],
                                          semg[b]).wait()

            def wait_add(b):
                pltpu.make_async_copy(m1[b], accum.at[ixd[b]], sema[b]).wait()

            issue_fills(0, 0)
            wait_idx(0)
            issue_gather(0)

            def body(jj, carry):
                for b in (0, 1):
                    j = 2 * jj + b
                    nb = 1 - b

                    @pl.when(j + 1 < SCHUNKS)
                    def _():
                        @pl.when(j >= 1)
                        def _():
                            wait_add(nb)
                        issue_fills(j + 1, nb)

                    wait_fills(b)
                    if kind is not None:
                        def edge(i, carry2):
                            us = plsc.load_gather(
                                rb[b],
                                [jnp.full((16,), i * 8 + kind, jnp.int32)])
                            for g in range(8):
                                sl = pl.ds(g * 16, 16)
                                m1[b][i, sl] = (m1[b][i, sl] * tr[b][i, sl]
                                                + us * m2[b][i, sl])
                            return carry2

                        lax.fori_loop(0, SK, edge, 0, unroll=4)
                    pltpu.async_copy(m1[b], accum.at[ixd[b]], sema[b], add=True)

                    @pl.when(j + 1 < SCHUNKS)
                    def _():
                        wait_idx(nb)
                        issue_gather(nb)
                return carry

            lax.fori_loop(0, SCHUNKS // 2, body, 0)
            wait_add(0)
            wait_add(1)
            plsc.subcore_barrier()
            pltpu.sync_copy(accum.at[pl.ds(sid * NPT, NPT)],
                            out.at[pl.ds(sid * NPT, NPT)])

        @pl.when(cid == 0)
        def _():
            job(kind0, tab0, m1a_hbm, out0)

        @pl.when(cid == 1)
        def _():
            job(kind1, tab1, m1b_hbm, out1)

    return _scatter


_scatter_call_1 = _make_scatter(None, 0)
_scatter_call_2 = _make_scatter(1, 2)


def _edge_math_kernel(r_ref, ssrc_ref,
                      cpad_ref, wpad_ref,
                      w_w1_ref, w_b1_ref, w_w2_ref, w_b2_ref,
                      ps_w1_ref, ps_b1_ref, ps_w2_ref, ps_b2_ref,
                      ds_ref, dv1_ref, dv2_ref):
    mask = (lax.broadcasted_iota(jnp.int32, (1, 8), 1) < 3).astype(jnp.float32)
    r = r_ref[...] * mask
    d2 = jnp.sum(r * r, axis=1, keepdims=True)
    d = jnp.maximum(jnp.sqrt(d2), 1e-6)
    x = d / CUTOFF
    env = jnp.where(x < 1.0, 0.5 * (jnp.cos(math.pi * x) + 1.0), 0.0)
    feat = jnp.exp(-wpad_ref[...] * (d - cpad_ref[...]) ** 2) * env
    h1 = jnp.dot(feat, w_w1_ref[...], preferred_element_type=jnp.float32)
    h1 = h1 + w_b1_ref[...]
    h1 = h1 * jax.nn.sigmoid(h1)
    W = jnp.dot(h1, w_w2_ref[...], preferred_element_type=jnp.float32)
    W = W + w_b2_ref[...]
    g1 = jnp.dot(ssrc_ref[...], ps_w1_ref[...], preferred_element_type=jnp.float32)
    g1 = g1 + ps_b1_ref[...]
    g1 = g1 * jax.nn.sigmoid(g1)
    phi = jnp.dot(g1, ps_w2_ref[...], preferred_element_type=jnp.float32)
    phi = phi + ps_b2_ref[...]
    sp = phi * W
    ds_ref[...] = sp[:, 0:H]
    dv1_ref[...] = sp[:, H:2 * H]
    dv2_ref[...] = sp[:, 2 * H:3 * H] / d


def _edge_math(r8, ssrc, cpad, wpad, w_w1p, w_b1, w_w2, w_b2,
               ps_w1, ps_b1, ps_w2, ps_b2):
    grid = (E // EDGE_BLK,)
    eblk = lambda w: pl.BlockSpec((EDGE_BLK, w), lambda i: (i, 0))
    full = lambda a: pl.BlockSpec(a.shape, lambda i: (0,) * a.ndim)
    return pl.pallas_call(
        _edge_math_kernel,
        grid=grid,
        in_specs=[eblk(8), eblk(H),
                  full(cpad), full(wpad),
                  full(w_w1p), full(w_b1), full(w_w2), full(w_b2),
                  full(ps_w1), full(ps_b1), full(ps_w2), full(ps_b2)],
        out_specs=[eblk(H), eblk(H), eblk(H)],
        out_shape=[
            jax.ShapeDtypeStruct((E, H), jnp.float32),
            jax.ShapeDtypeStruct((E, H), jnp.float32),
            jax.ShapeDtypeStruct((E, H), jnp.float32),
        ],
    )(r8, ssrc, cpad, wpad, w_w1p, w_b1, w_w2, w_b2,
      ps_w1, ps_b1, ps_w2, ps_b2)


def kernel(s, v, pos, edge_index, rbf_centers, rbf_widths,
           ps_w1, ps_b1, ps_w2, ps_b2, w_w1, w_b1, w_w2, w_b2):
    src = edge_index[0]
    dst = edge_index[1]

    px, py, pz = pos[:, 0], pos[:, 1], pos[:, 2]
    cpad = jnp.full((1, H), 1e9, jnp.float32).at[0, :R].set(rbf_centers)
    wpad = jnp.ones((1, H), jnp.float32).at[0, :R].set(jnp.abs(rbf_widths))
    w_w1p = jnp.zeros((H, H), jnp.float32).at[:R, :].set(w_w1)

    ssrc, r_flat = _sc_gather(s, px, py, pz, src, dst)
    r8 = r_flat.reshape(E, 8)

    dS, dV1, dV2s = _edge_math(
        r8, ssrc, cpad, wpad, w_w1p,
        w_b1.reshape(1, H), w_w2, w_b2.reshape(1, 3 * H),
        ps_w1, ps_b1.reshape(1, H), ps_w2, ps_b2.reshape(1, 3 * H))

    vt = jnp.transpose(v, (2, 0, 1))
    s_new, vo0 = _scatter_call_1(s, vt[0], dS, dV1, dV2s, r_flat, src, dst)
    vo1, vo2 = _scatter_call_2(vt[1], vt[2], dV1, dV1, dV2s, r_flat, src, dst)
    v_new = jnp.transpose(jnp.stack([vo0, vo1, vo2], axis=0), (1, 2, 0))
    return (s_new, v_new)
